# Initial kernel scaffold; baseline (speedup 1.0000x reference)
import jax
import jax.numpy as jnp
from jax.experimental import pallas as pl


def kernel(X, vertex, edges, W, eps):
    raise NotImplementedError("write your pallas kernel here")


# speedup vs baseline: 5.9027x; 5.9027x over previous
import functools

import jax
import jax.numpy as jnp
from jax import lax
from jax.experimental import pallas as pl
from jax.experimental.pallas import tpu as pltpu
from jax.experimental.pallas import tpu_sc as plsc

_N = 10000
_M = 20000
_P = 320000
_D = 128
_H = 64
_LANES = 128
_NS = 16
_NC = 2
_RPT = 160
_IDX_ROWS = _RPT * _NS
_P_PAD = _IDX_ROWS * _LANES
_MG = 20096
_NG = 10112
_MZ = _MG // _NS
_NZ = _NG // _NS
_CROWS = 16
_NCH = _RPT // _CROWS


def _sc_mesh():
    return plsc.VectorSubcoreMesh(
        core_axis_name="c", subcore_axis_name="s",
        num_cores=_NC, num_subcores=_NS)


@functools.partial(
    pl.kernel,
    out_type=(jax.ShapeDtypeStruct((2 * _MG, _H), jnp.float32),
              jax.ShapeDtypeStruct((_MG, 16), jnp.float32)),
    mesh=_sc_mesh(),
    compiler_params=pltpu.CompilerParams(use_tc_tiling_on_sc=False),
    scratch_types=[
        pltpu.VMEM_SHARED((_MG, _H), jnp.float32),
        pltpu.VMEM_SHARED((_MG, 16), jnp.float32),
        pltpu.VMEM((_CROWS, _LANES), jnp.int32),
        pltpu.VMEM((_CROWS, _LANES), jnp.int32),
        pltpu.VMEM((_LANES, _H), jnp.float32),
        pltpu.VMEM((_LANES, 16), jnp.float32),
        pltpu.SemaphoreType.DMA,
    ],
)
def _k1(xcat, vtx, edg, ones_h, z64, z16,
        sums_out, counts_out,
        sums_s, counts_s, vidx, eidx, rows, ones_v, gsem):
    cid = lax.axis_index("c")
    sid = lax.axis_index("s")
    pltpu.sync_copy(z64, sums_s.at[pl.ds(sid * _MZ, _MZ)])

    @pl.when(cid == 0)
    def _():
        pltpu.sync_copy(z16, counts_s.at[pl.ds(sid * _MZ, _MZ)])

    pltpu.sync_copy(ones_h, ones_v)
    plsc.subcore_barrier()

    def chunk(ci, carry):
        base = sid * _RPT + ci * _CROWS
        pltpu.sync_copy(vtx.at[pl.ds(cid * _IDX_ROWS + base, _CROWS)], vidx)
        pltpu.sync_copy(edg.at[pl.ds(base, _CROWS)], eidx)

        def step(j, c2):
            pltpu.async_copy(xcat.at[vidx.at[j]], rows, gsem).wait()
            pltpu.sync_copy(rows, sums_s.at[eidx.at[j]], add=True)

            @pl.when(cid == 0)
            def _():
                pltpu.sync_copy(ones_v, counts_s.at[eidx.at[j]], add=True)

            return c2

        lax.fori_loop(0, _CROWS, step, 0)
        return carry

    lax.fori_loop(0, _NCH, chunk, 0)
    plsc.subcore_barrier()

    pltpu.sync_copy(
        sums_s.at[pl.ds(sid * _MZ, _MZ)],
        sums_out.at[pl.ds(cid * _MG + sid * _MZ, _MZ)])

    @pl.when(cid == 0)
    def _():
        pltpu.sync_copy(
            counts_s.at[pl.ds(sid * _MZ, _MZ)],
            counts_out.at[pl.ds(sid * _MZ, _MZ)])


_BM2 = 2512


def _k2_body(s_ref, c_ref, o_ref):
    cnt = jnp.maximum(c_ref[:, 0:1], 1.0)
    o_ref[:, :] = s_ref[:, :] / cnt


def _k2(sums_cat, counts):
    return pl.pallas_call(
        _k2_body,
        grid=(2 * _MG // _BM2,),
        in_specs=[
            pl.BlockSpec((_BM2, _H), lambda i: (i, 0)),
            pl.BlockSpec((_BM2, 16), lambda i: (i % (_MG // _BM2), 0)),
        ],
        out_specs=pl.BlockSpec((_BM2, _H), lambda i: (i, 0)),
        out_shape=jax.ShapeDtypeStruct((2 * _MG, _H), jnp.float32),
    )(sums_cat, counts)


@functools.partial(
    pl.kernel,
    out_type=jax.ShapeDtypeStruct((2 * _NG, _H), jnp.float32),
    mesh=_sc_mesh(),
    compiler_params=pltpu.CompilerParams(use_tc_tiling_on_sc=False),
    scratch_types=[
        pltpu.VMEM_SHARED((_NG, _H), jnp.float32),
        pltpu.VMEM((_CROWS, _LANES), jnp.int32),
        pltpu.VMEM((_CROWS, _LANES), jnp.int32),
        pltpu.VMEM((_LANES, _H), jnp.float32),
        pltpu.SemaphoreType.DMA,
    ],
)
def _k3(xecat, edg, vtx, z64,
        xv_out,
        xv_s, eidx, vidx, rows, gsem):
    cid = lax.axis_index("c")
    sid = lax.axis_index("s")
    pltpu.sync_copy(z64.at[pl.ds(0, _NZ)], xv_s.at[pl.ds(sid * _NZ, _NZ)])
    plsc.subcore_barrier()

    def chunk(ci, carry):
        base = sid * _RPT + ci * _CROWS
        pltpu.sync_copy(edg.at[pl.ds(cid * _IDX_ROWS + base, _CROWS)], eidx)
        pltpu.sync_copy(vtx.at[pl.ds(base, _CROWS)], vidx)

        def step(j, c2):
            pltpu.async_copy(xecat.at[eidx.at[j]], rows, gsem).wait()
            pltpu.sync_copy(rows, xv_s.at[vidx.at[j]], add=True)
            return c2

        lax.fori_loop(0, _CROWS, step, 0)
        return carry

    lax.fori_loop(0, _NCH, chunk, 0)
    plsc.subcore_barrier()

    pltpu.sync_copy(
        xv_s.at[pl.ds(sid * _NZ, _NZ)],
        xv_out.at[pl.ds(cid * _NG + sid * _NZ, _NZ)])


_BN4 = 1000


def _k4_body(x_ref, lo_ref, hi_ref, w_ref, e_ref, o_ref):
    xn = e_ref[0, 0] * x_ref[:, :] + jnp.concatenate(
        [lo_ref[:, :], hi_ref[:, :]], axis=1)
    o_ref[:, :] = lax.dot_general(
        xn, w_ref[:, :], (((1,), (1,)), ((), ())),
        preferred_element_type=jnp.float32)


def _k4(X, xv_lo, xv_hi, W, epsp1):
    nb = _N // _BN4
    return pl.pallas_call(
        _k4_body,
        grid=(nb,),
        in_specs=[
            pl.BlockSpec((_BN4, _D), lambda i: (i, 0)),
            pl.BlockSpec((_BN4, _H), lambda i: (i, 0)),
            pl.BlockSpec((_BN4, _H), lambda i: (i, 0)),
            pl.BlockSpec((_D, _D), lambda i: (0, 0)),
            pl.BlockSpec(memory_space=pltpu.SMEM),
        ],
        out_specs=pl.BlockSpec((_BN4, _D), lambda i: (i, 0)),
        out_shape=jax.ShapeDtypeStruct((_N, _D), jnp.float32),
    )(X, xv_lo, xv_hi, W, epsp1)


def kernel(X, vertex, edges, W, eps):
    pad = _P_PAD - _P
    ar = jnp.arange(pad, dtype=jnp.int32)
    vtx_g = jnp.concatenate([vertex, (ar * 7919) % _N])
    vtx_g2 = jnp.concatenate([vtx_g, vtx_g + _N]).reshape(2 * _IDX_ROWS, _LANES)
    edg_s = jnp.concatenate([edges, _M + (ar % (_MG - _M))]).reshape(
        _IDX_ROWS, _LANES)
    edg_g = jnp.concatenate([edges, (ar * 7919) % _M])
    edg_g2 = jnp.concatenate([edg_g, edg_g + _MG]).reshape(
        2 * _IDX_ROWS, _LANES)
    vtx_s = jnp.concatenate([vertex, _N + (ar % (_NG - _N))]).reshape(
        _IDX_ROWS, _LANES)

    xcat = jnp.concatenate([X[:, :_H], X[:, _H:]], axis=0)
    ones_h = jnp.ones((_LANES, 16), jnp.float32)
    z64 = jnp.zeros((_MZ, _H), jnp.float32)
    z16 = jnp.zeros((_MZ, 16), jnp.float32)

    sums_cat, counts = _k1(xcat, vtx_g2, edg_s, ones_h, z64, z16)
    xe_cat = _k2(sums_cat, counts)
    xv_cat = _k3(xe_cat, edg_g2, vtx_s, z64)
    xv_lo = xv_cat[:_N]
    xv_hi = xv_cat[_NG:_NG + _N]
    epsp1 = (1.0 + eps).reshape(1, 1)
    return _k4(X, xv_lo, xv_hi, W, epsp1)


# speedup vs baseline: 8.7983x; 1.4906x over previous
import functools

import jax
import jax.numpy as jnp
from jax import lax
from jax.experimental import pallas as pl
from jax.experimental.pallas import tpu as pltpu
from jax.experimental.pallas import tpu_sc as plsc

_N = 10000
_M = 20000
_P = 320000
_D = 128
_H = 64
_LANES = 128
_NS = 16
_NC = 2
_RPT = 160
_IDX_ROWS = _RPT * _NS
_P_PAD = _IDX_ROWS * _LANES
_MG = 20096
_NG = 10112
_MZ = _MG // _NS
_NZ = _NG // _NS
_CROWS = 40
_NCH = _RPT // _CROWS


def _sc_mesh():
    return plsc.VectorSubcoreMesh(
        core_axis_name="c", subcore_axis_name="s",
        num_cores=_NC, num_subcores=_NS)


@functools.partial(
    pl.kernel,
    out_type=(jax.ShapeDtypeStruct((2 * _MG, _H), jnp.float32),
              jax.ShapeDtypeStruct((2 * _MG, 16), jnp.float32)),
    mesh=_sc_mesh(),
    compiler_params=pltpu.CompilerParams(use_tc_tiling_on_sc=False),
    scratch_types=[
        pltpu.VMEM_SHARED((_MG, _H), jnp.float32),
        pltpu.VMEM_SHARED((_MG, 16), jnp.float32),
        pltpu.VMEM((_CROWS, _LANES), jnp.int32),
        pltpu.VMEM((_CROWS, _LANES), jnp.int32),
        pltpu.VMEM((_LANES, _H), jnp.float32),
        pltpu.VMEM((_LANES, _H), jnp.float32),
        pltpu.VMEM((_LANES, 16), jnp.float32),
        pltpu.SemaphoreType.DMA,
        pltpu.SemaphoreType.DMA,
    ],
)
def _k1(xcat, vtx, edg, ones_h, z64, z16,
        sums_out, counts_out,
        sums_s, counts_s, vidx, eidx, rows0, rows1, ones_v, sem0, sem1):
    cid = lax.axis_index("c")
    sid = lax.axis_index("s")
    pltpu.sync_copy(z64, sums_s.at[pl.ds(sid * _MZ, _MZ)])
    pltpu.sync_copy(z16, counts_s.at[pl.ds(sid * _MZ, _MZ)])
    pltpu.sync_copy(ones_h, ones_v)
    plsc.subcore_barrier()

    for ci in range(_NCH):
        base = sid * _RPT + ci * _CROWS
        pltpu.sync_copy(vtx.at[pl.ds(cid * _IDX_ROWS + base, _CROWS)], vidx)
        pltpu.sync_copy(edg.at[pl.ds(base, _CROWS)], eidx)
        cc = ci // (_NCH // 2)
        pltpu.async_copy(xcat.at[vidx.at[0]], rows0, sem0)

        def body(jj, carry, _cc=cc):
            s0 = jj * 2
            s1 = s0 + 1
            pltpu.async_copy(xcat.at[vidx.at[s1]], rows1, sem1)
            pltpu.make_async_copy(xcat.at[pl.ds(0, _LANES)], rows0, sem0).wait()
            pltpu.sync_copy(rows0, sums_s.at[eidx.at[s0]], add=True)

            @pl.when(cid == _cc)
            def _():
                pltpu.sync_copy(ones_v, counts_s.at[eidx.at[s0]], add=True)

            nxt = jnp.minimum(s1 + 1, _CROWS - 1)
            pltpu.async_copy(xcat.at[vidx.at[nxt]], rows0, sem0)
            pltpu.make_async_copy(xcat.at[pl.ds(0, _LANES)], rows1, sem1).wait()
            pltpu.sync_copy(rows1, sums_s.at[eidx.at[s1]], add=True)

            @pl.when(cid == _cc)
            def _():
                pltpu.sync_copy(ones_v, counts_s.at[eidx.at[s1]], add=True)

            return carry

        lax.fori_loop(0, _CROWS // 2, body, 0)
        pltpu.make_async_copy(xcat.at[pl.ds(0, _LANES)], rows0, sem0).wait()

    plsc.subcore_barrier()
    pltpu.sync_copy(
        sums_s.at[pl.ds(sid * _MZ, _MZ)],
        sums_out.at[pl.ds(cid * _MG + sid * _MZ, _MZ)])
    pltpu.sync_copy(
        counts_s.at[pl.ds(sid * _MZ, _MZ)],
        counts_out.at[pl.ds(cid * _MG + sid * _MZ, _MZ)])


_BM2 = 2512


def _k2_body(s_ref, ca_ref, cb_ref, o_ref):
    cnt = jnp.maximum(ca_ref[:, 0:1] + cb_ref[:, 0:1], 1.0)
    o_ref[:, :] = s_ref[:, :] / cnt


def _k2(sums_cat, counts):
    nb = _MG // _BM2
    return pl.pallas_call(
        _k2_body,
        grid=(2 * _MG // _BM2,),
        in_specs=[
            pl.BlockSpec((_BM2, _H), lambda i: (i, 0)),
            pl.BlockSpec((_BM2, 16), lambda i, _nb=nb: (i % _nb, 0)),
            pl.BlockSpec((_BM2, 16), lambda i, _nb=nb: (_nb + i % _nb, 0)),
        ],
        out_specs=pl.BlockSpec((_BM2, _H), lambda i: (i, 0)),
        out_shape=jax.ShapeDtypeStruct((2 * _MG, _H), jnp.float32),
    )(sums_cat, counts, counts)


@functools.partial(
    pl.kernel,
    out_type=jax.ShapeDtypeStruct((2 * _NG, _H), jnp.float32),
    mesh=_sc_mesh(),
    compiler_params=pltpu.CompilerParams(use_tc_tiling_on_sc=False),
    scratch_types=[
        pltpu.VMEM_SHARED((_NG, _H), jnp.float32),
        pltpu.VMEM((_CROWS, _LANES), jnp.int32),
        pltpu.VMEM((_CROWS, _LANES), jnp.int32),
        pltpu.VMEM((_LANES, _H), jnp.float32),
        pltpu.VMEM((_LANES, _H), jnp.float32),
        pltpu.SemaphoreType.DMA,
        pltpu.SemaphoreType.DMA,
    ],
)
def _k3(xecat, edg, vtx, z64,
        xv_out,
        xv_s, eidx, vidx, rows0, rows1, sem0, sem1):
    cid = lax.axis_index("c")
    sid = lax.axis_index("s")
    pltpu.sync_copy(z64.at[pl.ds(0, _NZ)], xv_s.at[pl.ds(sid * _NZ, _NZ)])
    plsc.subcore_barrier()

    for ci in range(_NCH):
        base = sid * _RPT + ci * _CROWS
        pltpu.sync_copy(edg.at[pl.ds(cid * _IDX_ROWS + base, _CROWS)], eidx)
        pltpu.sync_copy(vtx.at[pl.ds(base, _CROWS)], vidx)
        pltpu.async_copy(xecat.at[eidx.at[0]], rows0, sem0)

        def body(jj, carry):
            s0 = jj * 2
            s1 = s0 + 1
            pltpu.async_copy(xecat.at[eidx.at[s1]], rows1, sem1)
            pltpu.make_async_copy(xecat.at[pl.ds(0, _LANES)], rows0, sem0).wait()
            pltpu.sync_copy(rows0, xv_s.at[vidx.at[s0]], add=True)
            nxt = jnp.minimum(s1 + 1, _CROWS - 1)
            pltpu.async_copy(xecat.at[eidx.at[nxt]], rows0, sem0)
            pltpu.make_async_copy(xecat.at[pl.ds(0, _LANES)], rows1, sem1).wait()
            pltpu.sync_copy(rows1, xv_s.at[vidx.at[s1]], add=True)
            return carry

        lax.fori_loop(0, _CROWS // 2, body, 0)
        pltpu.make_async_copy(xecat.at[pl.ds(0, _LANES)], rows0, sem0).wait()

    plsc.subcore_barrier()
    pltpu.sync_copy(
        xv_s.at[pl.ds(sid * _NZ, _NZ)],
        xv_out.at[pl.ds(cid * _NG + sid * _NZ, _NZ)])


_BN4 = 1000


def _k4_body(x_ref, lo_ref, hi_ref, w_ref, e_ref, o_ref):
    xn = e_ref[0, 0] * x_ref[:, :] + jnp.concatenate(
        [lo_ref[:, :], hi_ref[:, :]], axis=1)
    o_ref[:, :] = lax.dot_general(
        xn, w_ref[:, :], (((1,), (1,)), ((), ())),
        preferred_element_type=jnp.float32)


def _k4(X, xv_lo, xv_hi, W, epsp1):
    nb = _N // _BN4
    return pl.pallas_call(
        _k4_body,
        grid=(nb,),
        in_specs=[
            pl.BlockSpec((_BN4, _D), lambda i: (i, 0)),
            pl.BlockSpec((_BN4, _H), lambda i: (i, 0)),
            pl.BlockSpec((_BN4, _H), lambda i: (i, 0)),
            pl.BlockSpec((_D, _D), lambda i: (0, 0)),
            pl.BlockSpec(memory_space=pltpu.SMEM),
        ],
        out_specs=pl.BlockSpec((_BN4, _D), lambda i: (i, 0)),
        out_shape=jax.ShapeDtypeStruct((_N, _D), jnp.float32),
    )(X, xv_lo, xv_hi, W, epsp1)


def kernel(X, vertex, edges, W, eps):
    pad = _P_PAD - _P
    ar = jnp.arange(pad, dtype=jnp.int32)
    vtx_g = jnp.concatenate([vertex, (ar * 7919) % _N])
    vtx_g2 = jnp.concatenate([vtx_g, vtx_g + _N]).reshape(2 * _IDX_ROWS, _LANES)
    edg_s = jnp.concatenate([edges, _M + (ar % (_MG - _M))]).reshape(
        _IDX_ROWS, _LANES)
    edg_g = jnp.concatenate([edges, (ar * 7919) % _M])
    edg_g2 = jnp.concatenate([edg_g, edg_g + _MG]).reshape(
        2 * _IDX_ROWS, _LANES)
    vtx_s = jnp.concatenate([vertex, _N + (ar % (_NG - _N))]).reshape(
        _IDX_ROWS, _LANES)

    xcat = jnp.concatenate([X[:, :_H], X[:, _H:]], axis=0)
    ones_h = jnp.ones((_LANES, 16), jnp.float32)
    z64 = jnp.zeros((_MZ, _H), jnp.float32)
    z16 = jnp.zeros((_MZ, 16), jnp.float32)

    sums_cat, counts = _k1(xcat, vtx_g2, edg_s, ones_h, z64, z16)
    xe_cat = _k2(sums_cat, counts)
    xv_cat = _k3(xe_cat, edg_g2, vtx_s, z64)
    xv_lo = xv_cat[:_N]
    xv_hi = xv_cat[_NG:_NG + _N]
    epsp1 = (1.0 + eps).reshape(1, 1)
    return _k4(X, xv_lo, xv_hi, W, epsp1)


# speedup vs baseline: 10.1799x; 1.1570x over previous
import functools

import jax
import jax.numpy as jnp
from jax import lax
from jax.experimental import pallas as pl
from jax.experimental.pallas import tpu as pltpu
from jax.experimental.pallas import tpu_sc as plsc

_N = 10000
_M = 20000
_P = 320000
_D = 128
_H = 64
_LANES = 128
_NS = 16
_NC = 2
_RPT = 160
_IDX_ROWS = _RPT * _NS
_P_PAD = _IDX_ROWS * _LANES
_MG = 20096
_NG = 10112
_MZ = _MG // _NS
_NZ = _NG // _NS
_CROWS = 40
_NCH = _RPT // _CROWS
_CW = 8


def _sc_mesh():
    return plsc.VectorSubcoreMesh(
        core_axis_name="c", subcore_axis_name="s",
        num_cores=_NC, num_subcores=_NS)


@functools.partial(
    pl.kernel,
    out_type=(jax.ShapeDtypeStruct((2 * _MG, _H), jnp.float32),
              jax.ShapeDtypeStruct((2 * _MG, _CW), jnp.float32)),
    mesh=_sc_mesh(),
    compiler_params=pltpu.CompilerParams(use_tc_tiling_on_sc=False),
    scratch_types=[
        pltpu.VMEM_SHARED((_MG, _H), jnp.float32),
        pltpu.VMEM_SHARED((_MG, _CW), jnp.float32),
        pltpu.VMEM((_CROWS, _LANES), jnp.int32),
        pltpu.VMEM((_CROWS, _LANES), jnp.int32),
        pltpu.VMEM((_LANES, _H), jnp.float32),
        pltpu.VMEM((_LANES, _H), jnp.float32),
        pltpu.VMEM((_LANES, _H), jnp.float32),
        pltpu.VMEM((_LANES, _CW), jnp.float32),
        pltpu.SemaphoreType.DMA,
        pltpu.SemaphoreType.DMA,
        pltpu.SemaphoreType.DMA,
    ],
)
def _k1(xcat, vtx, edg, ones_h, z64, zc,
        sums_out, counts_out,
        sums_s, counts_s, vidx, eidx, rows0, rows1, rows2, ones_v,
        sem0, sem1, sem2):
    cid = lax.axis_index("c")
    sid = lax.axis_index("s")
    pltpu.sync_copy(z64, sums_s.at[pl.ds(sid * _MZ, _MZ)])
    pltpu.sync_copy(zc, counts_s.at[pl.ds(sid * _MZ, _MZ)])
    pltpu.sync_copy(ones_h, ones_v)
    plsc.subcore_barrier()

    rows = (rows0, rows1, rows2)
    sems = (sem0, sem1, sem2)
    for ci in range(_NCH):
        base = sid * _RPT + ci * _CROWS
        pltpu.sync_copy(vtx.at[pl.ds(cid * _IDX_ROWS + base, _CROWS)], vidx)
        pltpu.sync_copy(edg.at[pl.ds(base, _CROWS)], eidx)
        cc = ci // (_NCH // 2)
        pltpu.async_copy(xcat.at[vidx.at[0]], rows[0], sems[0])
        pltpu.async_copy(xcat.at[vidx.at[1]], rows[1], sems[1])

        def body(jj, carry, _cc=cc):
            for b in range(3):
                s = jj * 3 + b
                nxt = jnp.minimum(s + 2, _CROWS - 1)
                pltpu.async_copy(
                    xcat.at[vidx.at[nxt]], rows[(b + 2) % 3], sems[(b + 2) % 3])
                pltpu.make_async_copy(
                    xcat.at[pl.ds(0, _LANES)], rows[b], sems[b]).wait()
                pltpu.sync_copy(rows[b], sums_s.at[eidx.at[s]], add=True)

                @pl.when(cid == _cc)
                def _():
                    pltpu.sync_copy(ones_v, counts_s.at[eidx.at[s]], add=True)

            return carry

        lax.fori_loop(0, (_CROWS - 1) // 3, body, 0)
        pltpu.make_async_copy(xcat.at[pl.ds(0, _LANES)], rows[0], sems[0]).wait()
        pltpu.sync_copy(rows[0], sums_s.at[eidx.at[_CROWS - 1]], add=True)

        @pl.when(cid == cc)
        def _():
            pltpu.sync_copy(ones_v, counts_s.at[eidx.at[_CROWS - 1]], add=True)

        pltpu.make_async_copy(xcat.at[pl.ds(0, _LANES)], rows[1], sems[1]).wait()

    plsc.subcore_barrier()
    pltpu.sync_copy(
        sums_s.at[pl.ds(sid * _MZ, _MZ)],
        sums_out.at[pl.ds(cid * _MG + sid * _MZ, _MZ)])
    pltpu.sync_copy(
        counts_s.at[pl.ds(sid * _MZ, _MZ)],
        counts_out.at[pl.ds(cid * _MG + sid * _MZ, _MZ)])


_BM2 = 2512


def _k2_body(s_ref, ca_ref, cb_ref, o_ref):
    cnt = jnp.maximum(ca_ref[:, 0:1] + cb_ref[:, 0:1], 1.0)
    o_ref[:, :] = s_ref[:, :] / cnt


def _k2(sums_cat, counts):
    nb = _MG // _BM2
    return pl.pallas_call(
        _k2_body,
        grid=(2 * _MG // _BM2,),
        in_specs=[
            pl.BlockSpec((_BM2, _H), lambda i: (i, 0)),
            pl.BlockSpec((_BM2, _CW), lambda i, _nb=nb: (i % _nb, 0)),
            pl.BlockSpec((_BM2, _CW), lambda i, _nb=nb: (_nb + i % _nb, 0)),
        ],
        out_specs=pl.BlockSpec((_BM2, _H), lambda i: (i, 0)),
        out_shape=jax.ShapeDtypeStruct((2 * _MG, _H), jnp.float32),
    )(sums_cat, counts, counts)


@functools.partial(
    pl.kernel,
    out_type=jax.ShapeDtypeStruct((2 * _NG, _H), jnp.float32),
    mesh=_sc_mesh(),
    compiler_params=pltpu.CompilerParams(use_tc_tiling_on_sc=False),
    scratch_types=[
        pltpu.VMEM_SHARED((_NG, _H), jnp.float32),
        pltpu.VMEM((_CROWS, _LANES), jnp.int32),
        pltpu.VMEM((_CROWS, _LANES), jnp.int32),
        pltpu.VMEM((_LANES, _H), jnp.float32),
        pltpu.VMEM((_LANES, _H), jnp.float32),
        pltpu.VMEM((_LANES, _H), jnp.float32),
        pltpu.VMEM((_LANES, _H), jnp.float32),
        pltpu.SemaphoreType.DMA,
        pltpu.SemaphoreType.DMA,
        pltpu.SemaphoreType.DMA,
        pltpu.SemaphoreType.DMA,
    ],
)
def _k3(xecat, edg, vtx, z64,
        xv_out,
        xv_s, eidx, vidx, rows0, rows1, rows2, rows3,
        sem0, sem1, sem2, sem3):
    cid = lax.axis_index("c")
    sid = lax.axis_index("s")
    pltpu.sync_copy(z64.at[pl.ds(0, _NZ)], xv_s.at[pl.ds(sid * _NZ, _NZ)])
    plsc.subcore_barrier()

    rows = (rows0, rows1, rows2, rows3)
    sems = (sem0, sem1, sem2, sem3)
    for ci in range(_NCH):
        base = sid * _RPT + ci * _CROWS
        pltpu.sync_copy(edg.at[pl.ds(cid * _IDX_ROWS + base, _CROWS)], eidx)
        pltpu.sync_copy(vtx.at[pl.ds(base, _CROWS)], vidx)
        pltpu.async_copy(xecat.at[eidx.at[0]], rows[0], sems[0])
        pltpu.async_copy(xecat.at[eidx.at[1]], rows[1], sems[1])
        pltpu.async_copy(xecat.at[eidx.at[2]], rows[2], sems[2])

        def body(jj, carry):
            for b in range(4):
                s = jj * 4 + b
                nxt = jnp.minimum(s + 3, _CROWS - 1)
                pltpu.async_copy(
                    xecat.at[eidx.at[nxt]], rows[(b + 3) % 4], sems[(b + 3) % 4])
                pltpu.make_async_copy(
                    xecat.at[pl.ds(0, _LANES)], rows[b], sems[b]).wait()
                pltpu.sync_copy(rows[b], xv_s.at[vidx.at[s]], add=True)
            return carry

        lax.fori_loop(0, _CROWS // 4, body, 0)
        pltpu.make_async_copy(xecat.at[pl.ds(0, _LANES)], rows[0], sems[0]).wait()
        pltpu.make_async_copy(xecat.at[pl.ds(0, _LANES)], rows[1], sems[1]).wait()
        pltpu.make_async_copy(xecat.at[pl.ds(0, _LANES)], rows[2], sems[2]).wait()

    plsc.subcore_barrier()
    pltpu.sync_copy(
        xv_s.at[pl.ds(sid * _NZ, _NZ)],
        xv_out.at[pl.ds(cid * _NG + sid * _NZ, _NZ)])


_BN4 = 1000


def _k4_body(x_ref, lo_ref, hi_ref, w_ref, e_ref, o_ref):
    xn = e_ref[0, 0] * x_ref[:, :] + jnp.concatenate(
        [lo_ref[:, :], hi_ref[:, :]], axis=1)
    o_ref[:, :] = lax.dot_general(
        xn, w_ref[:, :], (((1,), (1,)), ((), ())),
        preferred_element_type=jnp.float32)


def _k4(X, xv_lo, xv_hi, W, epsp1):
    nb = _N // _BN4
    return pl.pallas_call(
        _k4_body,
        grid=(nb,),
        in_specs=[
            pl.BlockSpec((_BN4, _D), lambda i: (i, 0)),
            pl.BlockSpec((_BN4, _H), lambda i: (i, 0)),
            pl.BlockSpec((_BN4, _H), lambda i: (i, 0)),
            pl.BlockSpec((_D, _D), lambda i: (0, 0)),
            pl.BlockSpec(memory_space=pltpu.SMEM),
        ],
        out_specs=pl.BlockSpec((_BN4, _D), lambda i: (i, 0)),
        out_shape=jax.ShapeDtypeStruct((_N, _D), jnp.float32),
    )(X, xv_lo, xv_hi, W, epsp1)


def kernel(X, vertex, edges, W, eps):
    pad = _P_PAD - _P
    ar = jnp.arange(pad, dtype=jnp.int32)
    vtx_g = jnp.concatenate([vertex, (ar * 7919) % _N])
    vtx_g2 = jnp.concatenate([vtx_g, vtx_g + _N]).reshape(2 * _IDX_ROWS, _LANES)
    edg_s = jnp.concatenate([edges, _M + (ar % (_MG - _M))]).reshape(
        _IDX_ROWS, _LANES)
    edg_g = jnp.concatenate([edges, (ar * 7919) % _M])
    edg_g2 = jnp.concatenate([edg_g, edg_g + _MG]).reshape(
        2 * _IDX_ROWS, _LANES)
    vtx_s = jnp.concatenate([vertex, _N + (ar % (_NG - _N))]).reshape(
        _IDX_ROWS, _LANES)

    xcat = jnp.concatenate([X[:, :_H], X[:, _H:]], axis=0)
    ones_h = jnp.ones((_LANES, _CW), jnp.float32)
    z64 = jnp.zeros((_MZ, _H), jnp.float32)
    zc = jnp.zeros((_MZ, _CW), jnp.float32)

    sums_cat, counts = _k1(xcat, vtx_g2, edg_s, ones_h, z64, zc)
    xe_cat = _k2(sums_cat, counts)
    xv_cat = _k3(xe_cat, edg_g2, vtx_s, z64)
    xv_lo = xv_cat[:_N]
    xv_hi = xv_cat[_NG:_NG + _N]
    epsp1 = (1.0 + eps).reshape(1, 1)
    return _k4(X, xv_lo, xv_hi, W, epsp1)
